# baseline (speedup 1.0000x reference)
import functools

import jax
import jax.numpy as jnp
from jax import lax
from jax.experimental import pallas as pl
from jax.experimental.pallas import tpu as pltpu
from jax.experimental.pallas import tpu_sc as plsc

N = 10000
E = 320000
D = 128
NHOP = 10

NC = 2
NS = 16
HD = D // NC
NPAD = 10240
RPT = NPAD // NS
NV = RPT // 16
CHUNK = 128
SUP = 8
NSUP = 20
EPAD = NS * NSUP * SUP * CHUNK
QC = RPT // CHUNK


_mesh = plsc.VectorSubcoreMesh(
    core_axis_name="c", subcore_axis_name="s", num_cores=NC, num_subcores=NS)
_sc_params = pltpu.CompilerParams(
    needs_layout_passes=False, use_tc_tiling_on_sc=False)


def _rsqrt16(x):
  i = plsc.bitcast(x, jnp.int32)
  i = jnp.int32(0x5F3759DF) - (i >> 1)
  y = plsc.bitcast(i, jnp.float32)
  for _ in range(3):
    y = y * (1.5 - 0.5 * x * y * y)
  return y


def _zero_rows(buf, nrows):
  z = jnp.zeros((16,), jnp.float32)
  def row(r, _):
    for j in range(HD // 16):
      buf[r, pl.ds(j * 16, 16)] = z
    return 0
  lax.fori_loop(0, nrows, row, 0)


def _fill_flat(buf, nvec, value):
  v = jnp.full((16,), value, jnp.float32)
  def vec(i, _):
    buf[pl.ds(i * 16, 16)] = v
    return 0
  lax.fori_loop(0, nvec, vec, 0)


def _scale_block(buf, pref, base):
  def row(r, _):
    v = plsc.load_gather(pref, [jnp.full((16,), r, jnp.int32) + base])
    for j in range(HD // 16):
      buf[r, pl.ds(j * 16, 16)] = buf[r, pl.ds(j * 16, 16)] * v
    return 0
  lax.fori_loop(0, CHUNK, row, 0)


def _edge_pass(A, B, srcg, dstg, s, srcsb, dstsb, gbufs, gsems, ssems,
               isem_s, isem_d, look):
  nb = len(gbufs)
  sd = [None] * nb
  gd = [None] * nb
  idw = [None, None]
  idw[0] = (pltpu.async_copy(srcg.at[s].at[0], srcsb.at[0], isem_s),
            pltpu.async_copy(dstg.at[s].at[0], dstsb.at[0], isem_d))
  for u in range(NSUP):
    par = u % 2
    idw[par][0].wait()
    idw[par][1].wait()
    if u < NSUP - 1:
      npar = (u + 1) % 2
      idw[npar] = (
          pltpu.async_copy(srcg.at[s].at[u + 1], srcsb.at[npar], isem_s),
          pltpu.async_copy(dstg.at[s].at[u + 1], dstsb.at[npar], isem_d))
    S = srcsb.at[par]
    Dx = dstsb.at[par]
    for i in range(SUP):
      p = i % nb
      if sd[p] is not None:
        sd[p].wait()
        sd[p] = None
      gd[p] = pltpu.async_copy(A.at[S.at[i]], gbufs[p], gsems[p])
      if i >= look - 1:
        t = i - look + 1
        pt = t % nb
        gd[pt].wait()
        sd[pt] = pltpu.async_copy(gbufs[pt], B.at[Dx.at[t]], ssems[pt],
                                  add=True)
    for t in range(SUP - look + 1, SUP):
      pt = t % nb
      gd[pt].wait()
      sd[pt] = pltpu.async_copy(gbufs[pt], B.at[Dx.at[t]], ssems[pt],
                                add=True)
    for p in range(nb):
      if sd[p] is not None:
        sd[p].wait()
        sd[p] = None


def _rsqrt_slice(deg_sh, tmpf, dest, r0):
  pltpu.sync_copy(deg_sh.at[pl.ds(r0, RPT)], tmpf)
  def vec(v, _):
    sl = pl.ds(v * 16, 16)
    dest[sl] = _rsqrt16(jnp.maximum(tmpf[sl], 1.0))
    return 0
  lax.fori_loop(0, NV, vec, 0)


def _sc_stage1_body(xh, srcg, dstg, aggh, rin_o, rout_o, cmid_o,
                    A, B, dgo_sh, dgi_sh, srcsb, dstsb, g0, g1, g2, g3,
                    onesb, routp, rinp, tmpf, accb,
                    gs0, gs1, gs2, gs3, ss0, ss1, ss2, ss3,
                    is_s, is_d):
  c = lax.axis_index("c")
  s = lax.axis_index("s")
  r0 = s * RPT
  gbufs = (g0, g1, g2, g3)
  gsems = (gs0, gs1, gs2, gs3)
  ssems = (ss0, ss1, ss2, ss3)

  _fill_flat(onesb, CHUNK // 16, 1.0)
  _fill_flat(tmpf, NV, 0.0)
  pltpu.sync_copy(tmpf, dgo_sh.at[pl.ds(r0, RPT)])
  pltpu.sync_copy(tmpf, dgi_sh.at[pl.ds(r0, RPT)])
  plsc.subcore_barrier()

  def sup(u, _):
    pltpu.sync_copy(srcg.at[s].at[u], srcsb.at[0])
    pltpu.sync_copy(dstg.at[s].at[u], dstsb.at[0])
    def chunk(i, _):
      pltpu.sync_copy(onesb, dgo_sh.at[srcsb.at[0].at[i]], add=True)
      pltpu.sync_copy(onesb, dgi_sh.at[dstsb.at[0].at[i]], add=True)
      return 0
    lax.fori_loop(0, SUP, chunk, 0)
    return 0
  lax.fori_loop(0, NSUP, sup, 0)
  plsc.subcore_barrier()

  _rsqrt_slice(dgo_sh, tmpf, routp, r0)
  _rsqrt_slice(dgi_sh, tmpf, rinp, r0)

  @pl.when(c == 0)
  def _():
    pltpu.sync_copy(routp, rout_o.at[pl.ds(r0, RPT)])
    pltpu.sync_copy(rinp, rin_o.at[pl.ds(r0, RPT)])
    def cv(v, _):
      sl = pl.ds(v * 16, 16)
      accb[sl] = rinp[sl] * routp[sl]
      return 0
    lax.fori_loop(0, NV, cv, 0)
    pltpu.sync_copy(accb, cmid_o.at[pl.ds(r0, RPT)])

  _zero_rows(g2, CHUNK)
  for q in range(QC):
    blk = pl.ds(r0 + q * CHUNK, CHUNK)
    pltpu.sync_copy(xh.at[c].at[blk], g0)
    _scale_block(g0, routp, q * CHUNK)
    pltpu.sync_copy(g0, A.at[blk])
    pltpu.sync_copy(g2, B.at[blk])
  plsc.subcore_barrier()

  _edge_pass(A, B, srcg, dstg, s, srcsb, dstsb, gbufs, gsems, ssems,
             is_s, is_d, 3)
  plsc.subcore_barrier()
  for q in range(QC):
    blk = pl.ds(r0 + q * CHUNK, CHUNK)
    pltpu.sync_copy(B.at[blk], g0)
    _scale_block(g0, rinp, q * CHUNK)
    pltpu.sync_copy(g0, aggh.at[c].at[blk])


def _sc_hops_body(h1h, srcg, dstg, rin_i, rout_i, cmid_i, hh,
                  A, B, srcsb, dstsb, g0, g1, g2, g3, g4,
                  routp, rinp, cmp_,
                  gs0, gs1, gs2, gs3, gs4, ss0, ss1, ss2, ss3, ss4,
                  is_s, is_d):
  c = lax.axis_index("c")
  s = lax.axis_index("s")
  r0 = s * RPT
  gbufs = (g0, g1, g2, g3, g4)
  gsems = (gs0, gs1, gs2, gs3, gs4)
  ssems = (ss0, ss1, ss2, ss3, ss4)

  pltpu.sync_copy(rout_i.at[pl.ds(r0, RPT)], routp)
  pltpu.sync_copy(rin_i.at[pl.ds(r0, RPT)], rinp)
  pltpu.sync_copy(cmid_i.at[pl.ds(r0, RPT)], cmp_)

  _zero_rows(g2, CHUNK)
  for q in range(QC):
    blk = pl.ds(r0 + q * CHUNK, CHUNK)
    pltpu.sync_copy(h1h.at[c].at[blk], g0)
    _scale_block(g0, routp, q * CHUNK)
    pltpu.sync_copy(g0, A.at[blk])
    pltpu.sync_copy(g2, B.at[blk])
  plsc.subcore_barrier()

  def hop(k, _):
    _edge_pass(A, B, srcg, dstg, s, srcsb, dstsb, gbufs, gsems, ssems,
               is_s, is_d, 4)
    plsc.subcore_barrier()
    _zero_rows(g2, CHUNK)
    for q in range(QC):
      blk = pl.ds(r0 + q * CHUNK, CHUNK)
      pltpu.sync_copy(B.at[blk], g0)
      _scale_block(g0, cmp_, q * CHUNK)
      pltpu.sync_copy(g0, A.at[blk])
      pltpu.sync_copy(g2, B.at[blk])
    plsc.subcore_barrier()
    return 0
  lax.fori_loop(0, NHOP - 1, hop, 0)

  _edge_pass(A, B, srcg, dstg, s, srcsb, dstsb, gbufs, gsems, ssems,
             is_s, is_d, 4)
  plsc.subcore_barrier()
  for q in range(QC):
    blk = pl.ds(r0 + q * CHUNK, CHUNK)
    pltpu.sync_copy(B.at[blk], g0)
    _scale_block(g0, rinp, q * CHUNK)
    pltpu.sync_copy(h1h.at[c].at[blk], g1)
    def addr(r, _):
      for j in range(HD // 16):
        sl = pl.ds(j * 16, 16)
        g0[r, sl] = g0[r, sl] + g1[r, sl]
      return 0
    lax.fori_loop(0, CHUNK, addr, 0)
    pltpu.sync_copy(g0, hh.at[c].at[blk])


_DMA = pltpu.SemaphoreType.DMA

_sc_stage1 = functools.partial(
    pl.kernel,
    compiler_params=_sc_params,
    out_type=(
        jax.ShapeDtypeStruct((NC, NPAD, HD), jnp.float32),
        jax.ShapeDtypeStruct((NPAD,), jnp.float32),
        jax.ShapeDtypeStruct((NPAD,), jnp.float32),
        jax.ShapeDtypeStruct((NPAD,), jnp.float32),
    ),
    mesh=_mesh,
    scratch_types=[
        pltpu.VMEM_SHARED((NPAD, HD), jnp.float32),
        pltpu.VMEM_SHARED((NPAD, HD), jnp.float32),
        pltpu.VMEM_SHARED((NPAD,), jnp.float32),
        pltpu.VMEM_SHARED((NPAD,), jnp.float32),
        pltpu.VMEM((2, SUP, CHUNK), jnp.int32),
        pltpu.VMEM((2, SUP, CHUNK), jnp.int32),
        pltpu.VMEM((CHUNK, HD), jnp.float32),
        pltpu.VMEM((CHUNK, HD), jnp.float32),
        pltpu.VMEM((CHUNK, HD), jnp.float32),
        pltpu.VMEM((CHUNK, HD), jnp.float32),
        pltpu.VMEM((CHUNK,), jnp.float32),
        pltpu.VMEM((RPT,), jnp.float32),
        pltpu.VMEM((RPT,), jnp.float32),
        pltpu.VMEM((RPT,), jnp.float32),
        pltpu.VMEM((RPT,), jnp.float32),
        _DMA, _DMA, _DMA, _DMA, _DMA, _DMA, _DMA, _DMA, _DMA, _DMA,
    ])(_sc_stage1_body)

_sc_hops = functools.partial(
    pl.kernel,
    compiler_params=_sc_params,
    out_type=jax.ShapeDtypeStruct((NC, NPAD, HD), jnp.float32),
    mesh=_mesh,
    scratch_types=[
        pltpu.VMEM_SHARED((NPAD, HD), jnp.float32),
        pltpu.VMEM_SHARED((NPAD, HD), jnp.float32),
        pltpu.VMEM((2, SUP, CHUNK), jnp.int32),
        pltpu.VMEM((2, SUP, CHUNK), jnp.int32),
        pltpu.VMEM((CHUNK, HD), jnp.float32),
        pltpu.VMEM((CHUNK, HD), jnp.float32),
        pltpu.VMEM((CHUNK, HD), jnp.float32),
        pltpu.VMEM((CHUNK, HD), jnp.float32),
        pltpu.VMEM((CHUNK, HD), jnp.float32),
        pltpu.VMEM((RPT,), jnp.float32),
        pltpu.VMEM((RPT,), jnp.float32),
        pltpu.VMEM((RPT,), jnp.float32),
        _DMA, _DMA, _DMA, _DMA, _DMA, _DMA, _DMA, _DMA, _DMA, _DMA,
        _DMA, _DMA,
    ])(_sc_hops_body)


BLK = 1024


def _tc_enc_body(aL, aR, We, be, Wt, bt, h1h, ht):
  lo, hi = pl.ds(0, HD), pl.ds(HD, HD)
  pre = aL[...] @ We[lo, :] + aR[...] @ We[hi, :] + be[...]
  h1 = jnp.maximum(pre, 0.0)
  h1h[0, :, :] = h1[:, :HD]
  h1h[1, :, :] = h1[:, HD:]
  pre_t = aL[...] @ Wt[lo, :] + aR[...] @ Wt[hi, :] + bt[...]
  ht[...] = jnp.maximum(pre_t, 0.0)


def _tc_pred_body(hL, hR, W1, b1, ap, W2, b2, out):
  lo, hi = pl.ds(0, HD), pl.ds(HD, HD)
  t = hL[...] @ W1[lo, :] + hR[...] @ W1[hi, :] + b1[...]
  t = jnp.where(t >= 0.0, t, ap[...] * t)
  out[...] = t @ W2[...] + b2[...]


def _full(shape):
  return pl.BlockSpec(shape, lambda i: tuple(0 for _ in shape))


def _tc_enc(aggh, W_enc, b_enc, W_tgt, b_tgt):
  return pl.pallas_call(
      _tc_enc_body,
      grid=(NPAD // BLK,),
      in_specs=[
          pl.BlockSpec((BLK, HD), lambda i: (i, 0)),
          pl.BlockSpec((BLK, HD), lambda i: (i, 0)),
          _full((D, D)), _full((1, D)), _full((D, D)), _full((1, D)),
      ],
      out_specs=[
          pl.BlockSpec((NC, BLK, HD), lambda i: (0, i, 0)),
          pl.BlockSpec((BLK, D), lambda i: (i, 0)),
      ],
      out_shape=[
          jax.ShapeDtypeStruct((NC, NPAD, HD), jnp.float32),
          jax.ShapeDtypeStruct((NPAD, D), jnp.float32),
      ],
  )(aggh[0], aggh[1], W_enc, b_enc.reshape(1, D), W_tgt, b_tgt.reshape(1, D))


def _tc_pred(hh, W1, b1, prelu_a, W2, b2):
  return pl.pallas_call(
      _tc_pred_body,
      grid=(NPAD // BLK,),
      in_specs=[
          pl.BlockSpec((BLK, HD), lambda i: (i, 0)),
          pl.BlockSpec((BLK, HD), lambda i: (i, 0)),
          _full((D, D)), _full((1, D)), _full((1, 1)), _full((D, D)),
          _full((1, D)),
      ],
      out_specs=pl.BlockSpec((BLK, D), lambda i: (i, 0)),
      out_shape=jax.ShapeDtypeStruct((NPAD, D), jnp.float32),
  )(hh[0], hh[1], W1, b1.reshape(1, D), prelu_a.reshape(1, 1), W2,
    b2.reshape(1, D))


def kernel(x, edge_index, W_enc, b_enc, W_tgt, b_tgt, W1, b1, prelu_a, W2, b2):
  src = edge_index[0]
  dst = edge_index[1]
  srcg = jnp.pad(src, (0, EPAD - E), constant_values=N).reshape(
      NS, NSUP, SUP, CHUNK)
  dstg = jnp.pad(dst, (0, EPAD - E), constant_values=N).reshape(
      NS, NSUP, SUP, CHUNK)
  xh = jnp.pad(x, ((0, NPAD - N), (0, 0))).reshape(NPAD, NC, HD).transpose(
      1, 0, 2)

  aggh, rin, rout, cmid = _sc_stage1(xh, srcg, dstg)
  h1h, h_target = _tc_enc(aggh, W_enc, b_enc, W_tgt, b_tgt)
  hh = _sc_hops(h1h, srcg, dstg, rin, rout, cmid)
  h_pred = _tc_pred(hh, W1, b1, prelu_a, W2, b2)

  h = jnp.concatenate([hh[0, :N], hh[1, :N]], axis=1)
  return h, h_pred[:N], h_target[:N]


# speedup vs baseline: 1.2701x; 1.2701x over previous
import functools

import jax
import jax.numpy as jnp
from jax import lax
from jax.experimental import pallas as pl
from jax.experimental.pallas import tpu as pltpu
from jax.experimental.pallas import tpu_sc as plsc

N = 10000
E = 320000
D = 128
NHOP = 10

NC = 2
NS = 16
HD = D // NC
NPAD = 10240
RPT = NPAD // NS
NV = RPT // 16
CHUNK = 128
SUP = 16
NSUP = 10
EPAD = NS * NSUP * SUP * CHUNK
QC = RPT // CHUNK


_mesh = plsc.VectorSubcoreMesh(
    core_axis_name="c", subcore_axis_name="s", num_cores=NC, num_subcores=NS)
_sc_params = pltpu.CompilerParams(
    needs_layout_passes=False, use_tc_tiling_on_sc=False)


def _rsqrt16(x):
  i = plsc.bitcast(x, jnp.int32)
  i = jnp.int32(0x5F3759DF) - (i >> 1)
  y = plsc.bitcast(i, jnp.float32)
  for _ in range(3):
    y = y * (1.5 - 0.5 * x * y * y)
  return y


def _zero_rows(buf, nrows):
  z = jnp.zeros((16,), jnp.float32)
  def row(r, _):
    for j in range(HD // 16):
      buf[r, pl.ds(j * 16, 16)] = z
    return 0
  lax.fori_loop(0, nrows, row, 0)


def _fill_flat(buf, nvec, value):
  v = jnp.full((16,), value, jnp.float32)
  def vec(i, _):
    buf[pl.ds(i * 16, 16)] = v
    return 0
  lax.fori_loop(0, nvec, vec, 0)


def _scale_block(buf, pref, base):
  def row(r, _):
    v = plsc.load_gather(pref, [jnp.full((16,), r, jnp.int32) + base])
    for j in range(HD // 16):
      buf[r, pl.ds(j * 16, 16)] = buf[r, pl.ds(j * 16, 16)] * v
    return 0
  lax.fori_loop(0, CHUNK, row, 0)


def _edge_pass(A, B, srcg, dstg, s, srcsb, dstsb, gbufs, gsems, ssems,
               isem_s, isem_d, look, ring):
  nb = len(gbufs)
  sd = [None] * nb
  gd = [None] * nb
  idw = [None] * ring
  idw[0] = (pltpu.async_copy(srcg.at[s].at[0], srcsb.at[0], isem_s),
            pltpu.async_copy(dstg.at[s].at[0], dstsb.at[0], isem_d))
  for u in range(NSUP):
    m = u % ring
    idw[m][0].wait()
    idw[m][1].wait()
    if u < NSUP - 1:
      nm = (u + 1) % ring
      idw[nm] = (
          pltpu.async_copy(srcg.at[s].at[u + 1], srcsb.at[nm], isem_s),
          pltpu.async_copy(dstg.at[s].at[u + 1], dstsb.at[nm], isem_d))
    S = srcsb.at[m]
    Dx = dstsb.at[m]
    for i in range(SUP):
      p = i % nb
      if sd[p] is not None:
        sd[p].wait()
        sd[p] = None
      gd[p] = pltpu.async_copy(A.at[S.at[i]], gbufs[p], gsems[p])
      if i >= look - 1:
        t = i - look + 1
        pt = t % nb
        gd[pt].wait()
        sd[pt] = pltpu.async_copy(gbufs[pt], B.at[Dx.at[t]], ssems[pt],
                                  add=True)
    for t in range(SUP - look + 1, SUP):
      pt = t % nb
      gd[pt].wait()
      sd[pt] = pltpu.async_copy(gbufs[pt], B.at[Dx.at[t]], ssems[pt],
                                add=True)
    if ring == 2 or u == NSUP - 1:
      for p in range(nb):
        if sd[p] is not None:
          sd[p].wait()
          sd[p] = None


def _rsqrt_slice(deg_sh, tmpf, dest, r0):
  pltpu.sync_copy(deg_sh.at[pl.ds(r0, RPT)], tmpf)
  def vec(v, _):
    sl = pl.ds(v * 16, 16)
    dest[sl] = _rsqrt16(jnp.maximum(tmpf[sl], 1.0))
    return 0
  lax.fori_loop(0, NV, vec, 0)


def _sc_stage1_body(xh, srcg, dstg, aggh, rin_o, rout_o, cmid_o,
                    A, B, dgo_sh, dgi_sh, srcsb, dstsb, g0, g1, g2, g3,
                    onesb, routp, rinp, tmpf, accb,
                    gs0, gs1, gs2, gs3, ss0, ss1, ss2, ss3,
                    is_s, is_d):
  c = lax.axis_index("c")
  s = lax.axis_index("s")
  r0 = s * RPT
  gbufs = (g0, g1, g2, g3)
  gsems = (gs0, gs1, gs2, gs3)
  ssems = (ss0, ss1, ss2, ss3)

  _fill_flat(onesb, CHUNK // 16, 1.0)
  _fill_flat(tmpf, NV, 0.0)
  pltpu.sync_copy(tmpf, dgo_sh.at[pl.ds(r0, RPT)])
  pltpu.sync_copy(tmpf, dgi_sh.at[pl.ds(r0, RPT)])
  plsc.subcore_barrier()

  def sup(u, _):
    pltpu.sync_copy(srcg.at[s].at[u], srcsb.at[0])
    pltpu.sync_copy(dstg.at[s].at[u], dstsb.at[0])
    hd_ = [None] * 4
    for i in range(SUP):
      for tgt, buf, k in ((dgo_sh, srcsb, 0), (dgi_sh, dstsb, 1)):
        slot = (2 * i + k) % 4
        if hd_[slot] is not None:
          hd_[slot].wait()
        hd_[slot] = pltpu.async_copy(onesb, tgt.at[buf.at[0].at[i]],
                                     ssems[slot], add=True)
    for d in hd_:
      d.wait()
    return 0
  lax.fori_loop(0, NSUP, sup, 0)
  plsc.subcore_barrier()

  _rsqrt_slice(dgo_sh, tmpf, routp, r0)
  _rsqrt_slice(dgi_sh, tmpf, rinp, r0)

  @pl.when(c == 0)
  def _():
    pltpu.sync_copy(routp, rout_o.at[pl.ds(r0, RPT)])
    pltpu.sync_copy(rinp, rin_o.at[pl.ds(r0, RPT)])
    def cv(v, _):
      sl = pl.ds(v * 16, 16)
      accb[sl] = rinp[sl] * routp[sl]
      return 0
    lax.fori_loop(0, NV, cv, 0)
    pltpu.sync_copy(accb, cmid_o.at[pl.ds(r0, RPT)])

  _zero_rows(g2, CHUNK)
  for q in range(QC):
    blk = pl.ds(r0 + q * CHUNK, CHUNK)
    pltpu.sync_copy(xh.at[c].at[blk], g0)
    _scale_block(g0, routp, q * CHUNK)
    pltpu.sync_copy(g0, A.at[blk])
    pltpu.sync_copy(g2, B.at[blk])
  plsc.subcore_barrier()

  _edge_pass(A, B, srcg, dstg, s, srcsb, dstsb, gbufs, gsems, ssems,
             is_s, is_d, 3, 2)
  plsc.subcore_barrier()
  pltpu.sync_copy(B.at[pl.ds(r0, RPT)], aggh.at[c].at[pl.ds(r0, RPT)])


def _sc_hops_body(g0h, srcg, dstg, cmid_i, hh,
                  A, B, srcsb, dstsb, g0, g1, g2, g3, cmp_,
                  gs0, gs1, gs2, gs3, ss0, ss1, ss2, ss3,
                  is_s, is_d):
  c = lax.axis_index("c")
  s = lax.axis_index("s")
  r0 = s * RPT
  gbufs = (g0, g1, g2, g3)
  gsems = (gs0, gs1, gs2, gs3)
  ssems = (ss0, ss1, ss2, ss3)

  pltpu.sync_copy(cmid_i.at[pl.ds(r0, RPT)], cmp_)

  _zero_rows(g2, CHUNK)
  pltpu.sync_copy(g0h.at[c].at[pl.ds(r0, RPT)], A.at[pl.ds(r0, RPT)])
  for q in range(QC):
    pltpu.sync_copy(g2, B.at[pl.ds(r0 + q * CHUNK, CHUNK)])
  plsc.subcore_barrier()

  def hop(k, _):
    _edge_pass(A, B, srcg, dstg, s, srcsb, dstsb, gbufs, gsems, ssems,
               is_s, is_d, 3, 3)
    plsc.subcore_barrier()
    _zero_rows(g2, CHUNK)
    rb = (g0, g1)
    blks = [pl.ds(r0 + q * CHUNK, CHUNK) for q in range(QC)]
    rd = [None, None]
    wr = [None, None]
    zr = [None, None]
    rd[0] = pltpu.async_copy(B.at[blks[0]], g0, gsems[0])
    for q in range(QC):
      p = q % 2
      np_ = (q + 1) % 2
      if q < QC - 1:
        if wr[np_] is not None:
          wr[np_].wait()
        rd[np_] = pltpu.async_copy(B.at[blks[q + 1]], rb[np_], gsems[np_])
      rd[p].wait()
      _scale_block(rb[p], cmp_, q * CHUNK)
      wr[p] = pltpu.async_copy(rb[p], A.at[blks[q]], ssems[p])
      if zr[p] is not None:
        zr[p].wait()
      zr[p] = pltpu.async_copy(g2, B.at[blks[q]], ssems[2 + p])
    for d in wr + zr:
      if d is not None:
        d.wait()
    plsc.subcore_barrier()
    return 0
  lax.fori_loop(0, NHOP - 1, hop, 0)

  _edge_pass(A, B, srcg, dstg, s, srcsb, dstsb, gbufs, gsems, ssems,
             is_s, is_d, 3, 3)
  plsc.subcore_barrier()
  pltpu.sync_copy(B.at[pl.ds(r0, RPT)], hh.at[c].at[pl.ds(r0, RPT)])


_DMA = pltpu.SemaphoreType.DMA

_sc_stage1 = functools.partial(
    pl.kernel,
    compiler_params=_sc_params,
    out_type=(
        jax.ShapeDtypeStruct((NC, NPAD, HD), jnp.float32),
        jax.ShapeDtypeStruct((NPAD,), jnp.float32),
        jax.ShapeDtypeStruct((NPAD,), jnp.float32),
        jax.ShapeDtypeStruct((NPAD,), jnp.float32),
    ),
    mesh=_mesh,
    scratch_types=[
        pltpu.VMEM_SHARED((NPAD, HD), jnp.float32),
        pltpu.VMEM_SHARED((NPAD, HD), jnp.float32),
        pltpu.VMEM_SHARED((NPAD,), jnp.float32),
        pltpu.VMEM_SHARED((NPAD,), jnp.float32),
        pltpu.VMEM((2, SUP, CHUNK), jnp.int32),
        pltpu.VMEM((2, SUP, CHUNK), jnp.int32),
        pltpu.VMEM((CHUNK, HD), jnp.float32),
        pltpu.VMEM((CHUNK, HD), jnp.float32),
        pltpu.VMEM((CHUNK, HD), jnp.float32),
        pltpu.VMEM((CHUNK, HD), jnp.float32),
        pltpu.VMEM((CHUNK,), jnp.float32),
        pltpu.VMEM((RPT,), jnp.float32),
        pltpu.VMEM((RPT,), jnp.float32),
        pltpu.VMEM((RPT,), jnp.float32),
        pltpu.VMEM((RPT,), jnp.float32),
        _DMA, _DMA, _DMA, _DMA, _DMA, _DMA, _DMA, _DMA, _DMA, _DMA,
    ])(_sc_stage1_body)

_sc_hops = functools.partial(
    pl.kernel,
    compiler_params=_sc_params,
    out_type=jax.ShapeDtypeStruct((NC, NPAD, HD), jnp.float32),
    mesh=_mesh,
    scratch_types=[
        pltpu.VMEM_SHARED((NPAD, HD), jnp.float32),
        pltpu.VMEM_SHARED((NPAD, HD), jnp.float32),
        pltpu.VMEM((3, SUP, CHUNK), jnp.int32),
        pltpu.VMEM((3, SUP, CHUNK), jnp.int32),
        pltpu.VMEM((CHUNK, HD), jnp.float32),
        pltpu.VMEM((CHUNK, HD), jnp.float32),
        pltpu.VMEM((CHUNK, HD), jnp.float32),
        pltpu.VMEM((CHUNK, HD), jnp.float32),
        pltpu.VMEM((RPT,), jnp.float32),
        _DMA, _DMA, _DMA, _DMA, _DMA, _DMA, _DMA, _DMA, _DMA, _DMA,
    ])(_sc_hops_body)


BLK = 1024


def _tc_enc_body(aL, aR, rin, rout, We, be, Wt, bt, h1h, g0h, ht):
  lo, hi = pl.ds(0, HD), pl.ds(HD, HD)
  sL = aL[...] * rin[...]
  sR = aR[...] * rin[...]
  pre = sL @ We[lo, :] + sR @ We[hi, :] + be[...]
  h1 = jnp.maximum(pre, 0.0)
  h1h[0, :, :] = h1[:, :HD]
  h1h[1, :, :] = h1[:, HD:]
  g0 = h1 * rout[...]
  g0h[0, :, :] = g0[:, :HD]
  g0h[1, :, :] = g0[:, HD:]
  pre_t = sL @ Wt[lo, :] + sR @ Wt[hi, :] + bt[...]
  ht[...] = jnp.maximum(pre_t, 0.0)


def _tc_pred_body(sL, sR, h1L, h1R, rin, W1, b1, ap, W2, b2, out_h, out_p):
  hL = h1L[...] + sL[...] * rin[...]
  hR = h1R[...] + sR[...] * rin[...]
  out_h[:, :HD] = hL
  out_h[:, HD:] = hR
  lo, hi = pl.ds(0, HD), pl.ds(HD, HD)
  t = hL @ W1[lo, :] + hR @ W1[hi, :] + b1[...]
  t = jnp.where(t >= 0.0, t, ap[...] * t)
  out_p[...] = t @ W2[...] + b2[...]


def _full(shape):
  return pl.BlockSpec(shape, lambda i: tuple(0 for _ in shape))


def _tc_enc(aggh, rin, rout, W_enc, b_enc, W_tgt, b_tgt):
  return pl.pallas_call(
      _tc_enc_body,
      grid=(NPAD // BLK,),
      in_specs=[
          pl.BlockSpec((BLK, HD), lambda i: (i, 0)),
          pl.BlockSpec((BLK, HD), lambda i: (i, 0)),
          pl.BlockSpec((BLK, 1), lambda i: (i, 0)),
          pl.BlockSpec((BLK, 1), lambda i: (i, 0)),
          _full((D, D)), _full((1, D)), _full((D, D)), _full((1, D)),
      ],
      out_specs=[
          pl.BlockSpec((NC, BLK, HD), lambda i: (0, i, 0)),
          pl.BlockSpec((NC, BLK, HD), lambda i: (0, i, 0)),
          pl.BlockSpec((BLK, D), lambda i: (i, 0)),
      ],
      out_shape=[
          jax.ShapeDtypeStruct((NC, NPAD, HD), jnp.float32),
          jax.ShapeDtypeStruct((NC, NPAD, HD), jnp.float32),
          jax.ShapeDtypeStruct((NPAD, D), jnp.float32),
      ],
  )(aggh[0], aggh[1], rin.reshape(NPAD, 1), rout.reshape(NPAD, 1),
    W_enc, b_enc.reshape(1, D), W_tgt, b_tgt.reshape(1, D))


def _tc_pred(hh, h1h, rin, W1, b1, prelu_a, W2, b2):
  return pl.pallas_call(
      _tc_pred_body,
      grid=(NPAD // BLK,),
      in_specs=[
          pl.BlockSpec((BLK, HD), lambda i: (i, 0)),
          pl.BlockSpec((BLK, HD), lambda i: (i, 0)),
          pl.BlockSpec((BLK, HD), lambda i: (i, 0)),
          pl.BlockSpec((BLK, HD), lambda i: (i, 0)),
          pl.BlockSpec((BLK, 1), lambda i: (i, 0)),
          _full((D, D)), _full((1, D)), _full((1, 1)), _full((D, D)),
          _full((1, D)),
      ],
      out_specs=[
          pl.BlockSpec((BLK, D), lambda i: (i, 0)),
          pl.BlockSpec((BLK, D), lambda i: (i, 0)),
      ],
      out_shape=[
          jax.ShapeDtypeStruct((NPAD, D), jnp.float32),
          jax.ShapeDtypeStruct((NPAD, D), jnp.float32),
      ],
  )(hh[0], hh[1], h1h[0], h1h[1], rin.reshape(NPAD, 1),
    W1, b1.reshape(1, D), prelu_a.reshape(1, 1), W2, b2.reshape(1, D))


def kernel(x, edge_index, W_enc, b_enc, W_tgt, b_tgt, W1, b1, prelu_a, W2, b2):
  src = edge_index[0]
  dst = edge_index[1]
  srcg = jnp.pad(src, (0, EPAD - E), constant_values=N).reshape(
      NS, NSUP, SUP, CHUNK)
  dstg = jnp.pad(dst, (0, EPAD - E), constant_values=N).reshape(
      NS, NSUP, SUP, CHUNK)
  xh = jnp.pad(x, ((0, NPAD - N), (0, 0))).reshape(NPAD, NC, HD).transpose(
      1, 0, 2)

  aggh, rin, rout, cmid = _sc_stage1(xh, srcg, dstg)
  h1h, g0h, h_target = _tc_enc(aggh, rin, rout, W_enc, b_enc, W_tgt, b_tgt)
  hh = _sc_hops(g0h, srcg, dstg, cmid)
  h, h_pred = _tc_pred(hh, h1h, rin, W1, b1, prelu_a, W2, b2)

  return h[:N], h_pred[:N], h_target[:N]
